# baseline (device time: 102273 ns/iter reference)
import jax
import jax.numpy as jnp
from jax import lax
from jax.experimental import pallas as pl
from jax.experimental.pallas import tpu as pltpu

N_DEV = 8
B = 2
SQ = 256
SKV = 256
HQ = 4
DH = 64
BLK = 64
D_MODEL = 512


def kernel(x, Wq, K_ext, V_ext, Wo):
    my = lax.axis_index("i")
    h0 = my * HQ
    K_my = lax.dynamic_slice_in_dim(K_ext, h0, HQ, axis=2).reshape(B, SKV, HQ * DH)
    V_my = lax.dynamic_slice_in_dim(V_ext, h0, HQ, axis=2).reshape(B, SKV, HQ * DH)

    def body(x_ref, wq_ref, k_ref, v_ref, wo_ref, out_ref,
             comm_ref, send_sems, recv_sems):
        my_pos = lax.axis_index("i")
        left = lax.rem(my_pos + N_DEV - 1, N_DEV)
        right = lax.rem(my_pos + 1, N_DEV)

        barrier_sem = pltpu.get_barrier_semaphore()
        for nbr in (left, right):
            pl.semaphore_signal(
                barrier_sem, inc=1,
                device_id=(nbr,), device_id_type=pl.DeviceIdType.MESH,
            )
        pl.semaphore_wait(barrier_sem, 2)

        qb = lax.broadcasted_iota(jnp.int32, (SQ, SKV), 0) // BLK
        kb = lax.broadcasted_iota(jnp.int32, (SQ, SKV), 1) // BLK
        mask = kb <= qb

        for b in range(B):
            q = jnp.dot(x_ref[b], wq_ref[:, :],
                        preferred_element_type=jnp.float32)
            ctx_cols = []
            for h in range(HQ):
                qh = q[:, h * DH:(h + 1) * DH]
                kh = k_ref[b][:, h * DH:(h + 1) * DH]
                vh = v_ref[b][:, h * DH:(h + 1) * DH]
                s = lax.dot_general(
                    qh, kh, (((1,), (1,)), ((), ())),
                    preferred_element_type=jnp.float32) * 0.125
                s = jnp.where(mask, s, -1e9)
                m = jnp.max(s, axis=1, keepdims=True)
                w = jnp.exp(s - m)
                w = w / jnp.sum(w, axis=1, keepdims=True)
                ctx_cols.append(jnp.dot(w, vh,
                                        preferred_element_type=jnp.float32))
            ctx = jnp.concatenate(ctx_cols, axis=1)
            pb = jnp.dot(ctx, wo_ref[:, :],
                         preferred_element_type=jnp.float32)
            out_ref[b] = pb
            comm_ref[0, b] = pb

        for hop in range(N_DEV - 1):
            rdma = pltpu.make_async_remote_copy(
                src_ref=comm_ref.at[hop],
                dst_ref=comm_ref.at[hop + 1],
                send_sem=send_sems.at[hop],
                recv_sem=recv_sems.at[hop],
                device_id=(right,),
                device_id_type=pl.DeviceIdType.MESH,
            )
            rdma.start()
            rdma.wait()
            out_ref[:, :, :] = out_ref[:, :, :] + comm_ref[hop + 1]

    return pl.pallas_call(
        body,
        out_shape=jax.ShapeDtypeStruct((B, SQ, D_MODEL), jnp.float32),
        in_specs=[pl.BlockSpec(memory_space=pltpu.VMEM)] * 5,
        out_specs=pl.BlockSpec(memory_space=pltpu.VMEM),
        scratch_shapes=[
            pltpu.VMEM((N_DEV, B, SQ, D_MODEL), jnp.float32),
            pltpu.SemaphoreType.DMA((N_DEV - 1,)),
            pltpu.SemaphoreType.DMA((N_DEV - 1,)),
        ],
        compiler_params=pltpu.CompilerParams(collective_id=0),
    )(x, Wq, K_my, V_my, Wo)


# device time: 41427 ns/iter; 2.4688x vs baseline; 2.4688x over previous
import jax
import jax.numpy as jnp
from jax import lax
from jax.experimental import pallas as pl
from jax.experimental.pallas import tpu as pltpu

N_DEV = 8
B = 2
SQ = 256
SKV = 256
HQ = 4
DH = 64
BLK = 64
D_MODEL = 512

ROWS = B * SQ
CH = ROWS // N_DEV


def kernel(x, Wq, K_ext, V_ext, Wo):
    my = lax.axis_index("i")
    h0 = my * HQ
    K_my = lax.dynamic_slice_in_dim(K_ext, h0, HQ, axis=2).reshape(B, SKV, HQ * DH)
    V_my = lax.dynamic_slice_in_dim(V_ext, h0, HQ, axis=2).reshape(B, SKV, HQ * DH)

    def body(x_ref, wq_ref, k_ref, v_ref, wo_ref, out_ref,
             acc_ref, rs_buf, rs_send, rs_recv, ag_send, ag_recv):
        my_pos = lax.axis_index("i")
        bits = [jnp.bitwise_and(lax.shift_right_logical(my_pos, r), 1)
                for r in range(3)]
        partners = [jnp.bitwise_xor(my_pos, 1 << r) for r in range(3)]

        barrier_sem = pltpu.get_barrier_semaphore()
        for p in partners:
            pl.semaphore_signal(
                barrier_sem, inc=1,
                device_id=(p,), device_id_type=pl.DeviceIdType.MESH,
            )
        pl.semaphore_wait(barrier_sem, 3)

        qb = lax.broadcasted_iota(jnp.int32, (SQ, SKV), 0) // BLK
        kb = lax.broadcasted_iota(jnp.int32, (SQ, SKV), 1) // BLK
        mask = kb <= qb

        for b in range(B):
            q = jnp.dot(x_ref[b], wq_ref[:, :],
                        preferred_element_type=jnp.float32)
            ctx_cols = []
            for h in range(HQ):
                qh = q[:, h * DH:(h + 1) * DH]
                kh = k_ref[b][:, h * DH:(h + 1) * DH]
                vh = v_ref[b][:, h * DH:(h + 1) * DH]
                s = lax.dot_general(
                    qh, kh, (((1,), (1,)), ((), ())),
                    preferred_element_type=jnp.float32) * 0.125
                s = jnp.where(mask, s, -1e9)
                m = jnp.max(s, axis=1, keepdims=True)
                w = jnp.exp(s - m)
                w = w / jnp.sum(w, axis=1, keepdims=True)
                ctx_cols.append(jnp.dot(w, vh,
                                        preferred_element_type=jnp.float32))
            ctx = jnp.concatenate(ctx_cols, axis=1)
            pb = jnp.dot(ctx, wo_ref[:, :],
                         preferred_element_type=jnp.float32)
            acc_ref[b * SQ:(b + 1) * SQ, :] = pb

        base = jnp.int32(0)
        rs_off = 0
        for r, n in zip(range(3), (ROWS // 2, ROWS // 4, ROWS // 8)):
            keep = base + bits[r] * n
            send = base + (1 - bits[r]) * n
            rdma = pltpu.make_async_remote_copy(
                src_ref=acc_ref.at[pl.ds(send, n)],
                dst_ref=rs_buf.at[pl.ds(rs_off, n)],
                send_sem=rs_send.at[r],
                recv_sem=rs_recv.at[r],
                device_id=(partners[r],),
                device_id_type=pl.DeviceIdType.MESH,
            )
            rdma.start()
            rdma.wait()
            acc_ref[pl.ds(keep, n), :] = (
                acc_ref[pl.ds(keep, n), :] + rs_buf[rs_off:rs_off + n, :]
            )
            base = keep
            rs_off += n

        for r, n in zip((2, 1, 0), (ROWS // 8, ROWS // 4, ROWS // 2)):
            rdma = pltpu.make_async_remote_copy(
                src_ref=acc_ref.at[pl.ds(base, n)],
                dst_ref=acc_ref.at[pl.ds(base, n)],
                send_sem=ag_send.at[r],
                recv_sem=ag_recv.at[r],
                device_id=(partners[r],),
                device_id_type=pl.DeviceIdType.MESH,
            )
            rdma.start()
            rdma.wait()
            base = base - bits[r] * n

        for b in range(B):
            out_ref[b] = acc_ref[b * SQ:(b + 1) * SQ, :]

    return pl.pallas_call(
        body,
        out_shape=jax.ShapeDtypeStruct((B, SQ, D_MODEL), jnp.float32),
        in_specs=[pl.BlockSpec(memory_space=pltpu.VMEM)] * 5,
        out_specs=pl.BlockSpec(memory_space=pltpu.VMEM),
        scratch_shapes=[
            pltpu.VMEM((ROWS, D_MODEL), jnp.float32),
            pltpu.VMEM((ROWS // 2 + ROWS // 4 + ROWS // 8, D_MODEL),
                       jnp.float32),
            pltpu.SemaphoreType.DMA((3,)),
            pltpu.SemaphoreType.DMA((3,)),
            pltpu.SemaphoreType.DMA((3,)),
            pltpu.SemaphoreType.DMA((3,)),
        ],
        compiler_params=pltpu.CompilerParams(collective_id=0),
    )(x, Wq, K_my, V_my, Wo)


# device time: 34991 ns/iter; 2.9228x vs baseline; 1.1839x over previous
import jax
import jax.numpy as jnp
from jax import lax
from jax.experimental import pallas as pl
from jax.experimental.pallas import tpu as pltpu

N_DEV = 8
B = 2
SQ = 256
SKV = 256
HQ = 4
DH = 64
BLK = 64
D_MODEL = 512

ROWS = B * SQ
CH = ROWS // N_DEV


def kernel(x, Wq, K_ext, V_ext, Wo):
    my = lax.axis_index("i")
    h0 = my * HQ
    K_my = lax.dynamic_slice_in_dim(K_ext, h0, HQ, axis=2).reshape(B * SKV, HQ * DH)
    V_my = lax.dynamic_slice_in_dim(V_ext, h0, HQ, axis=2).reshape(B * SKV, HQ * DH)
    x_flat = x.reshape(ROWS, D_MODEL)

    def body(x_ref, wq_ref, k_ref, v_ref, wo_ref, out_ref,
             rs_buf, rs_send, rs_recv, ag_send, ag_recv):
        my_pos = lax.axis_index("i")
        bits = [jnp.bitwise_and(lax.shift_right_logical(my_pos, r), 1)
                for r in range(3)]
        partners = [jnp.bitwise_xor(my_pos, 1 << r) for r in range(3)]

        barrier_sem = pltpu.get_barrier_semaphore()
        for d in range(1, N_DEV):
            pl.semaphore_signal(
                barrier_sem, inc=1,
                device_id=(jnp.bitwise_xor(my_pos, d),),
                device_id_type=pl.DeviceIdType.MESH,
            )
        pl.semaphore_wait(barrier_sem, N_DEV - 1)

        qb = lax.broadcasted_iota(jnp.int32, (SQ, SKV), 0) // BLK
        kb = lax.broadcasted_iota(jnp.int32, (SQ, SKV), 1) // BLK
        mask = kb <= qb

        first = 1 - bits[0]
        rdma0 = None
        for p in range(2):
            bb = jnp.bitwise_xor(first, p)
            row0 = bb * SQ
            q = jnp.dot(x_ref[pl.ds(row0, SQ), :], wq_ref[:, :],
                        preferred_element_type=jnp.float32)
            kf = k_ref[pl.ds(bb * SKV, SKV), :]
            vf = v_ref[pl.ds(bb * SKV, SKV), :]
            ctx_cols = []
            for h in range(HQ):
                qh = q[:, h * DH:(h + 1) * DH]
                kh = kf[:, h * DH:(h + 1) * DH]
                vh = vf[:, h * DH:(h + 1) * DH]
                s = lax.dot_general(
                    qh, kh, (((1,), (1,)), ((), ())),
                    preferred_element_type=jnp.float32) * 0.125
                s = jnp.where(mask, s, -1e9)
                m = jnp.max(s, axis=1, keepdims=True)
                w = jnp.exp(s - m)
                w = w / jnp.sum(w, axis=1, keepdims=True)
                ctx_cols.append(jnp.dot(w, vh,
                                        preferred_element_type=jnp.float32))
            ctx = jnp.concatenate(ctx_cols, axis=1)
            pb = jnp.dot(ctx, wo_ref[:, :],
                         preferred_element_type=jnp.float32)
            out_ref[pl.ds(row0, SQ), :] = pb
            if p == 0:
                rdma0 = pltpu.make_async_remote_copy(
                    src_ref=out_ref.at[pl.ds(row0, ROWS // 2)],
                    dst_ref=rs_buf.at[pl.ds(0, ROWS // 2)],
                    send_sem=rs_send.at[0],
                    recv_sem=rs_recv.at[0],
                    device_id=(partners[0],),
                    device_id_type=pl.DeviceIdType.MESH,
                )
                rdma0.start()

        rdma0.wait()
        base = bits[0] * (ROWS // 2)
        out_ref[pl.ds(base, ROWS // 2), :] = (
            out_ref[pl.ds(base, ROWS // 2), :] + rs_buf[0:ROWS // 2, :]
        )
        rs_off = ROWS // 2
        for r, n in zip((1, 2), (ROWS // 4, ROWS // 8)):
            keep = base + bits[r] * n
            send = base + (1 - bits[r]) * n
            rdma = pltpu.make_async_remote_copy(
                src_ref=out_ref.at[pl.ds(send, n)],
                dst_ref=rs_buf.at[pl.ds(rs_off, n)],
                send_sem=rs_send.at[r],
                recv_sem=rs_recv.at[r],
                device_id=(partners[r],),
                device_id_type=pl.DeviceIdType.MESH,
            )
            rdma.start()
            rdma.wait()
            out_ref[pl.ds(keep, n), :] = (
                out_ref[pl.ds(keep, n), :] + rs_buf[rs_off:rs_off + n, :]
            )
            base = keep
            rs_off += n

        ag = []
        for d in range(1, N_DEV):
            rdma = pltpu.make_async_remote_copy(
                src_ref=out_ref.at[pl.ds(base, CH)],
                dst_ref=out_ref.at[pl.ds(base, CH)],
                send_sem=ag_send.at[d - 1],
                recv_sem=ag_recv.at[d - 1],
                device_id=(jnp.bitwise_xor(my_pos, d),),
                device_id_type=pl.DeviceIdType.MESH,
            )
            rdma.start()
            ag.append(rdma)
        for rdma in ag:
            rdma.wait()

    out = pl.pallas_call(
        body,
        out_shape=jax.ShapeDtypeStruct((ROWS, D_MODEL), jnp.float32),
        in_specs=[pl.BlockSpec(memory_space=pltpu.VMEM)] * 5,
        out_specs=pl.BlockSpec(memory_space=pltpu.VMEM),
        scratch_shapes=[
            pltpu.VMEM((ROWS // 2 + ROWS // 4 + ROWS // 8, D_MODEL),
                       jnp.float32),
            pltpu.SemaphoreType.DMA((3,)),
            pltpu.SemaphoreType.DMA((3,)),
            pltpu.SemaphoreType.DMA((N_DEV - 1,)),
            pltpu.SemaphoreType.DMA((N_DEV - 1,)),
        ],
        compiler_params=pltpu.CompilerParams(collective_id=0),
    )(x_flat, Wq, K_my, V_my, Wo)
    return out.reshape(B, SQ, D_MODEL)


# device time: 30485 ns/iter; 3.3549x vs baseline; 1.1478x over previous
import jax
import jax.numpy as jnp
from jax import lax
from jax.experimental import pallas as pl
from jax.experimental.pallas import tpu as pltpu

N_DEV = 8
B = 2
SQ = 256
SKV = 256
HQ = 4
DH = 64
BLK = 64
D_MODEL = 512

ROWS = B * SQ
CH = ROWS // N_DEV


def _chunk_base(t):
    b0 = jnp.bitwise_and(t, 1)
    b1 = jnp.bitwise_and(lax.shift_right_logical(t, 1), 1)
    b2 = jnp.bitwise_and(lax.shift_right_logical(t, 2), 1)
    return b0 * (ROWS // 2) + b1 * (ROWS // 4) + b2 * (ROWS // 8)


def kernel(x, Wq, K_ext, V_ext, Wo):
    my = lax.axis_index("i")
    h0 = my * HQ
    K_my = lax.dynamic_slice_in_dim(K_ext, h0, HQ, axis=2).reshape(B * SKV, HQ * DH)
    V_my = lax.dynamic_slice_in_dim(V_ext, h0, HQ, axis=2).reshape(B * SKV, HQ * DH)
    x_flat = x.reshape(ROWS, D_MODEL)

    def body(x_ref, wq_ref, k_ref, v_ref, wo_ref, out_ref,
             rs_buf, rs_send, rs_recv, ag_send, ag_recv):
        my_pos = lax.axis_index("i")

        barrier_sem = pltpu.get_barrier_semaphore()
        for d in range(1, N_DEV):
            pl.semaphore_signal(
                barrier_sem, inc=1,
                device_id=(jnp.bitwise_xor(my_pos, d),),
                device_id_type=pl.DeviceIdType.MESH,
            )
        pl.semaphore_wait(barrier_sem, N_DEV - 1)

        rs = []
        for d in range(1, N_DEV):
            t = jnp.bitwise_xor(my_pos, d)
            rs.append((
                pltpu.make_async_remote_copy(
                    src_ref=out_ref.at[pl.ds(_chunk_base(t), CH)],
                    dst_ref=rs_buf.at[pl.ds((d - 1) * CH, CH)],
                    send_sem=rs_send.at[d - 1],
                    recv_sem=rs_recv.at[d - 1],
                    device_id=(t,),
                    device_id_type=pl.DeviceIdType.MESH,
                ),
                jnp.bitwise_and(t, 1),
            ))

        qb = lax.broadcasted_iota(jnp.int32, (SQ, SKV), 0) // BLK
        kb = lax.broadcasted_iota(jnp.int32, (SQ, SKV), 1) // BLK
        mask = kb <= qb

        for p in range(B):
            row0 = p * SQ
            q = jnp.dot(x_ref[row0:row0 + SQ, :], wq_ref[:, :],
                        preferred_element_type=jnp.float32)
            kf = k_ref[p * SKV:(p + 1) * SKV, :]
            vf = v_ref[p * SKV:(p + 1) * SKV, :]
            ctx_cols = []
            for h in range(HQ):
                qh = q[:, h * DH:(h + 1) * DH]
                kh = kf[:, h * DH:(h + 1) * DH]
                vh = vf[:, h * DH:(h + 1) * DH]
                s = lax.dot_general(
                    qh, kh, (((1,), (1,)), ((), ())),
                    preferred_element_type=jnp.float32) * 0.125
                s = jnp.where(mask, s, -1e9)
                m = jnp.max(s, axis=1, keepdims=True)
                w = jnp.exp(s - m)
                w = w / jnp.sum(w, axis=1, keepdims=True)
                ctx_cols.append(jnp.dot(w, vh,
                                        preferred_element_type=jnp.float32))
            ctx = jnp.concatenate(ctx_cols, axis=1)
            pb = jnp.dot(ctx, wo_ref[:, :],
                         preferred_element_type=jnp.float32)
            out_ref[row0:row0 + SQ, :] = pb

            for rdma, t_batch in rs:
                @pl.when(t_batch == p)
                def _(rdma=rdma):
                    rdma.start()

        base = _chunk_base(my_pos)
        for rdma, _ in rs:
            rdma.wait()
        red = out_ref[pl.ds(base, CH), :]
        for j in range(N_DEV - 1):
            red = red + rs_buf[j * CH:(j + 1) * CH, :]
        out_ref[pl.ds(base, CH), :] = red

        ag = []
        for d in range(1, N_DEV):
            rdma = pltpu.make_async_remote_copy(
                src_ref=out_ref.at[pl.ds(base, CH)],
                dst_ref=out_ref.at[pl.ds(base, CH)],
                send_sem=ag_send.at[d - 1],
                recv_sem=ag_recv.at[d - 1],
                device_id=(jnp.bitwise_xor(my_pos, d),),
                device_id_type=pl.DeviceIdType.MESH,
            )
            rdma.start()
            ag.append(rdma)
        for rdma in ag:
            rdma.wait()

    out = pl.pallas_call(
        body,
        out_shape=jax.ShapeDtypeStruct((ROWS, D_MODEL), jnp.float32),
        in_specs=[pl.BlockSpec(memory_space=pltpu.VMEM)] * 5,
        out_specs=pl.BlockSpec(memory_space=pltpu.VMEM),
        scratch_shapes=[
            pltpu.VMEM(((N_DEV - 1) * CH, D_MODEL), jnp.float32),
            pltpu.SemaphoreType.DMA((N_DEV - 1,)),
            pltpu.SemaphoreType.DMA((N_DEV - 1,)),
            pltpu.SemaphoreType.DMA((N_DEV - 1,)),
            pltpu.SemaphoreType.DMA((N_DEV - 1,)),
        ],
        compiler_params=pltpu.CompilerParams(collective_id=0),
    )(x_flat, Wq, K_my, V_my, Wo)
    return out.reshape(B, SQ, D_MODEL)
